# baseline (speedup 1.0000x reference)
import functools

import jax
import jax.numpy as jnp
from jax import lax
from jax.experimental import pallas as pl
from jax.experimental.pallas import tpu as pltpu
from jax.experimental.pallas import tpu_sc as plsc

_N_IN = 256
_N_H1 = 1024
_N_H2 = 1024
_N_OUT = 128
_FAN = 16
_BATCH = 2048

_NC = 2
_NS = 16
_NW = _NC * _NS
_L = 16


def _sc_densify(src1, w1, src2, w2, src3, w3):
    mesh = plsc.VectorSubcoreMesh(core_axis_name="c", subcore_axis_name="s")

    @functools.partial(
        pl.kernel,
        mesh=mesh,
        compiler_params=pltpu.CompilerParams(needs_layout_passes=False,
                                             skip_device_barrier=True),
        out_type=[
            jax.ShapeDtypeStruct((_N_H1, _N_IN), jnp.float32),
            jax.ShapeDtypeStruct((_N_H2, _N_H1), jnp.float32),
            jax.ShapeDtypeStruct((_N_OUT, _N_H2), jnp.float32),
        ],
        scratch_types=[
            pltpu.VMEM((_N_H2 // _NW * _FAN,), jnp.int32),
            pltpu.VMEM((_N_H2 // _NW * _FAN,), jnp.float32),
            pltpu.VMEM((_N_H1 // _NW, _N_IN), jnp.float32),
            pltpu.VMEM((_N_H2 // _NW, _N_H1), jnp.float32),
            pltpu.VMEM((_N_OUT // _NW, _N_H2), jnp.float32),
            pltpu.VMEM((_N_H1 // _NW * _FAN,), jnp.int32),
            pltpu.VMEM((_N_H1 // _NW * _FAN,), jnp.float32),
            pltpu.VMEM((_N_OUT // _NW * _FAN,), jnp.int32),
            pltpu.VMEM((_N_OUT // _NW * _FAN,), jnp.float32),
            pltpu.SemaphoreType.DMA,
            pltpu.SemaphoreType.DMA,
            pltpu.SemaphoreType.DMA,
            pltpu.SemaphoreType.DMA,
        ],
    )
    def k(src1_h, w1_h, src2_h, w2_h, src3_h, w3_h, o1, o2, o3,
          src2_v, w2_v, acc1, acc2, acc3, src1_v, w1_v, src3_v, w3_v,
          in1_sem, in2_sem, in3_sem, out_sem):
        wid = lax.axis_index("s") * _NC + lax.axis_index("c")
        lanes = lax.iota(jnp.int32, _L)
        zeros16 = jnp.zeros((_L,), jnp.float32)

        layers = (
            (src2_h, w2_h, o2, acc2, src2_v, w2_v, in2_sem, _N_H2, _N_H1),
            (src1_h, w1_h, o1, acc1, src1_v, w1_v, in1_sem, _N_H1, _N_IN),
            (src3_h, w3_h, o3, acc3, src3_v, w3_v, in3_sem, _N_OUT, _N_H2),
        )

        in_dmas = []
        for (src_h, w_h, _, _, src_v, w_v, sem, n_nodes, _) in layers:
            n_e = (n_nodes // _NW) * _FAN
            base_e = wid * n_e
            in_dmas.append(pltpu.async_copy(
                src_h.at[pl.ds(base_e, n_e)], src_v, sem))
            in_dmas.append(pltpu.async_copy(
                w_h.at[pl.ds(base_e, n_e)], w_v, sem))

        for (_, _, _, acc, _, _, _, n_nodes, d) in layers:
            npw = n_nodes // _NW

            def zero_body(j, _, acc=acc, d=d):
                for c in range(d // _L):
                    acc[j, pl.ds(c * _L, _L)] = zeros16
                return 0
            lax.fori_loop(0, npw, zero_body, 0)

        out_dmas = []
        for li, (_, _, o_h, acc, src_v, w_v, sem, n_nodes, d) in enumerate(
                layers):
            in_dmas[2 * li].wait()
            in_dmas[2 * li + 1].wait()
            del sem
            npw = n_nodes // _NW
            nblocks = max(1, npw // _L)
            for nb in range(nblocks):
                local_nodes = lanes + nb * _L
                mask = local_nodes < npw if npw < _L else None
                for i in range(_FAN):
                    eidx = local_nodes * _FAN + i
                    cols = plsc.load_gather(src_v, [eidx])
                    vals = plsc.load_gather(w_v, [eidx])
                    if mask is None:
                        plsc.addupdate_scatter(acc, [local_nodes, cols], vals)
                    else:
                        plsc.addupdate_scatter(acc, [local_nodes, cols], vals,
                                               mask=mask)
                rows = min(_L, npw)
                out_dmas.append(pltpu.async_copy(
                    acc.at[pl.ds(nb * _L, rows)],
                    o_h.at[pl.ds(wid * npw + nb * _L, rows)], out_sem))
        for dma in out_dmas:
            dma.wait()

    return k(src1, w1, src2, w2, src3, w3)


def _tc_forward(x, w1t, w2t, w3t):
    dn = (((1,), (1,)), ((), ()))

    def body(x_h, w1_h, w2_h, w3_h, o_h,
             xv, w1v, w2v, w3v, h1v, h2v, ov,
             s1, s2, s3, sx, so):
        cp_x = pltpu.make_async_copy(x_h, xv, sx)
        cp_w1 = pltpu.make_async_copy(w1_h, w1v, s1)
        cp_w2 = pltpu.make_async_copy(w2_h, w2v, s2)
        cp_w3 = pltpu.make_async_copy(w3_h, w3v, s3)
        cp_x.start()
        cp_w1.start()
        cp_w2.start()
        cp_w3.start()
        cp_x.wait()
        cp_w1.wait()
        h1v[...] = jnp.tanh(lax.dot_general(
            xv[...], w1v[...], dn, preferred_element_type=jnp.float32))
        cp_w2.wait()
        h2v[...] = jnp.tanh(lax.dot_general(
            h1v[...], w2v[...], dn, preferred_element_type=jnp.float32))
        cp_w3.wait()
        ov[...] = jnp.tanh(lax.dot_general(
            h2v[...], w3v[...], dn, preferred_element_type=jnp.float32))
        cp_o = pltpu.make_async_copy(ov, o_h, so)
        cp_o.start()
        cp_o.wait()

    return pl.pallas_call(
        body,
        in_specs=[
            pl.BlockSpec(memory_space=pltpu.HBM),
            pl.BlockSpec(memory_space=pltpu.HBM),
            pl.BlockSpec(memory_space=pltpu.HBM),
            pl.BlockSpec(memory_space=pltpu.HBM),
        ],
        out_specs=pl.BlockSpec(memory_space=pltpu.HBM),
        out_shape=jax.ShapeDtypeStruct((_BATCH, _N_OUT), jnp.float32),
        scratch_shapes=[
            pltpu.VMEM((_BATCH, _N_IN), jnp.float32),
            pltpu.VMEM((_N_H1, _N_IN), jnp.float32),
            pltpu.VMEM((_N_H2, _N_H1), jnp.float32),
            pltpu.VMEM((_N_OUT, _N_H2), jnp.float32),
            pltpu.VMEM((_BATCH, _N_H1), jnp.float32),
            pltpu.VMEM((_BATCH, _N_H2), jnp.float32),
            pltpu.VMEM((_BATCH, _N_OUT), jnp.float32),
            pltpu.SemaphoreType.DMA,
            pltpu.SemaphoreType.DMA,
            pltpu.SemaphoreType.DMA,
            pltpu.SemaphoreType.DMA,
            pltpu.SemaphoreType.DMA,
        ],
    )(x, w1t, w2t, w3t)


def kernel(x, w1, w2, w3, src1, dst1, src2, dst2, src3, dst3):
    del dst1, dst2, dst3
    w1t, w2t, w3t = _sc_densify(src1, w1, src2, w2, src3, w3)
    return _tc_forward(x, w1t, w2t, w3t)


# speedup vs baseline: 1.0359x; 1.0359x over previous
import functools

import jax
import jax.numpy as jnp
from jax import lax
from jax.experimental import pallas as pl
from jax.experimental.pallas import tpu as pltpu
from jax.experimental.pallas import tpu_sc as plsc

_N_IN = 256
_N_H1 = 1024
_N_H2 = 1024
_N_OUT = 128
_FAN = 16
_BATCH = 2048

_NC = 2
_NS = 16
_NW = _NC * _NS
_L = 16


def _sc_densify(src1, w1, src2, w2, src3, w3):
    mesh = plsc.VectorSubcoreMesh(core_axis_name="c", subcore_axis_name="s")

    @functools.partial(
        pl.kernel,
        mesh=mesh,
        compiler_params=pltpu.CompilerParams(needs_layout_passes=False,
                                             skip_device_barrier=True),
        out_type=[
            jax.ShapeDtypeStruct((_N_H1, _N_IN), jnp.float32),
            jax.ShapeDtypeStruct((_N_H2, _N_H1), jnp.float32),
            jax.ShapeDtypeStruct((_N_OUT, _N_H2), jnp.float32),
        ],
        scratch_types=[
            pltpu.VMEM((_N_H2 // _NW * _FAN,), jnp.int32),
            pltpu.VMEM((_N_H2 // _NW * _FAN,), jnp.float32),
            pltpu.VMEM((_N_H1 // _NW, _N_IN), jnp.float32),
            pltpu.VMEM((_N_H2 // _NW, _N_H1), jnp.float32),
            pltpu.VMEM((_N_OUT // _NW, _N_H2), jnp.float32),
            pltpu.VMEM((_N_H1 // _NW * _FAN,), jnp.int32),
            pltpu.VMEM((_N_H1 // _NW * _FAN,), jnp.float32),
            pltpu.VMEM((_N_OUT // _NW * _FAN,), jnp.int32),
            pltpu.VMEM((_N_OUT // _NW * _FAN,), jnp.float32),
            pltpu.SemaphoreType.DMA,
            pltpu.SemaphoreType.DMA,
            pltpu.SemaphoreType.DMA,
            pltpu.SemaphoreType.DMA,
        ],
    )
    def k(src1_h, w1_h, src2_h, w2_h, src3_h, w3_h, o1, o2, o3,
          src2_v, w2_v, acc1, acc2, acc3, src1_v, w1_v, src3_v, w3_v,
          in1_sem, in2_sem, in3_sem, out_sem):
        wid = lax.axis_index("s") * _NC + lax.axis_index("c")
        lanes = lax.iota(jnp.int32, _L)
        zeros16 = jnp.zeros((_L,), jnp.float32)

        layers = (
            (src2_h, w2_h, o2, acc2, src2_v, w2_v, in2_sem, _N_H2, _N_H1),
            (src1_h, w1_h, o1, acc1, src1_v, w1_v, in1_sem, _N_H1, _N_IN),
            (src3_h, w3_h, o3, acc3, src3_v, w3_v, in3_sem, _N_OUT, _N_H2),
        )

        in_dmas = []
        for (src_h, w_h, _, _, src_v, w_v, sem, n_nodes, _) in layers:
            n_e = (n_nodes // _NW) * _FAN
            base_e = wid * n_e
            in_dmas.append(pltpu.async_copy(
                src_h.at[pl.ds(base_e, n_e)], src_v, sem))
            in_dmas.append(pltpu.async_copy(
                w_h.at[pl.ds(base_e, n_e)], w_v, sem))

        for (_, _, _, acc, _, _, _, n_nodes, d) in layers:
            npw = n_nodes // _NW

            def zero_body(j, _, acc=acc, d=d):
                for c in range(d // _L):
                    acc[j, pl.ds(c * _L, _L)] = zeros16
                return 0
            lax.fori_loop(0, npw, zero_body, 0)

        out_dmas = []
        for li, (_, _, o_h, acc, src_v, w_v, sem, n_nodes, d) in enumerate(
                layers):
            in_dmas[2 * li].wait()
            in_dmas[2 * li + 1].wait()
            del sem
            npw = n_nodes // _NW
            nblocks = max(1, npw // _L)
            for nb in range(nblocks):
                local_nodes = lanes + nb * _L
                mask = local_nodes < npw if npw < _L else None
                for i in range(_FAN):
                    eidx = local_nodes * _FAN + i
                    cols = plsc.load_gather(src_v, [eidx])
                    vals = plsc.load_gather(w_v, [eidx])
                    if mask is None:
                        plsc.addupdate_scatter(acc, [local_nodes, cols], vals)
                    else:
                        plsc.addupdate_scatter(acc, [local_nodes, cols], vals,
                                               mask=mask)
                rows = min(_L, npw)
                out_dmas.append(pltpu.async_copy(
                    acc.at[pl.ds(nb * _L, rows)],
                    o_h.at[pl.ds(wid * npw + nb * _L, rows)], out_sem))
        for dma in out_dmas:
            dma.wait()

    return k(src1, w1, src2, w2, src3, w3)


def _tc_forward(x, w1t, w2t, w3t):
    bm = 1024
    dn = (((1,), (1,)), ((), ()))

    def body(x_ref, w1_ref, w2_ref, w3_ref, o_ref):
        h1 = jnp.tanh(lax.dot_general(x_ref[...], w1_ref[...], dn,
                                      preferred_element_type=jnp.float32))
        h2 = jnp.tanh(lax.dot_general(h1, w2_ref[...], dn,
                                      preferred_element_type=jnp.float32))
        o_ref[...] = jnp.tanh(lax.dot_general(h2, w3_ref[...], dn,
                                              preferred_element_type=jnp.float32))

    return pl.pallas_call(
        body,
        grid=(_BATCH // bm,),
        in_specs=[
            pl.BlockSpec((bm, _N_IN), lambda i: (i, 0)),
            pl.BlockSpec((_N_H1, _N_IN), lambda i: (0, 0)),
            pl.BlockSpec((_N_H2, _N_H1), lambda i: (0, 0)),
            pl.BlockSpec((_N_OUT, _N_H2), lambda i: (0, 0)),
        ],
        out_specs=pl.BlockSpec((bm, _N_OUT), lambda i: (i, 0)),
        out_shape=jax.ShapeDtypeStruct((_BATCH, _N_OUT), jnp.float32),
    )(x, w1t, w2t, w3t)


def kernel(x, w1, w2, w3, src1, dst1, src2, dst2, src3, dst3):
    del dst1, dst2, dst3
    w1t, w2t, w3t = _sc_densify(src1, w1, src2, w2, src3, w3)
    return _tc_forward(x, w1t, w2t, w3t)


# speedup vs baseline: 1.0372x; 1.0013x over previous
import functools

import jax
import jax.numpy as jnp
from jax import lax
from jax.experimental import pallas as pl
from jax.experimental.pallas import tpu as pltpu
from jax.experimental.pallas import tpu_sc as plsc

_N_IN = 256
_N_H1 = 1024
_N_H2 = 1024
_N_OUT = 128
_FAN = 16
_BATCH = 2048

_NC = 2
_NS = 16
_NW = _NC * _NS
_L = 16


def _sc_densify(src1, w1, src2, w2, src3, w3):
    mesh = plsc.VectorSubcoreMesh(core_axis_name="c", subcore_axis_name="s")

    @functools.partial(
        pl.kernel,
        mesh=mesh,
        compiler_params=pltpu.CompilerParams(needs_layout_passes=False,
                                             skip_device_barrier=True),
        out_type=[
            jax.ShapeDtypeStruct((_N_H1, _N_IN), jnp.float32),
            jax.ShapeDtypeStruct((_N_H2, _N_H1), jnp.float32),
            jax.ShapeDtypeStruct((_N_OUT, _N_H2), jnp.float32),
        ],
        scratch_types=[
            pltpu.VMEM((_N_H2 // _NW * _FAN,), jnp.int32),
            pltpu.VMEM((_N_H2 // _NW * _FAN,), jnp.float32),
            pltpu.VMEM((_N_H1 // _NW, _N_IN), jnp.float32),
            pltpu.VMEM((_N_H2 // _NW, _N_H1), jnp.float32),
            pltpu.VMEM((_N_OUT // _NW, _N_H2), jnp.float32),
            pltpu.VMEM((_N_H1 // _NW * _FAN,), jnp.int32),
            pltpu.VMEM((_N_H1 // _NW * _FAN,), jnp.float32),
            pltpu.VMEM((_N_OUT // _NW * _FAN,), jnp.int32),
            pltpu.VMEM((_N_OUT // _NW * _FAN,), jnp.float32),
            pltpu.SemaphoreType.DMA,
            pltpu.SemaphoreType.DMA,
            pltpu.SemaphoreType.DMA,
            pltpu.SemaphoreType.DMA,
        ],
    )
    def k(src1_h, w1_h, src2_h, w2_h, src3_h, w3_h, o1, o2, o3,
          src2_v, w2_v, acc1, acc2, acc3, src1_v, w1_v, src3_v, w3_v,
          in1_sem, in2_sem, in3_sem, out_sem):
        wid = lax.axis_index("s") * _NC + lax.axis_index("c")
        lanes = lax.iota(jnp.int32, _L)
        zeros16 = jnp.zeros((_L,), jnp.float32)

        layers = (
            (src2_h, w2_h, o2, acc2, src2_v, w2_v, in2_sem, _N_H2, _N_H1),
            (src1_h, w1_h, o1, acc1, src1_v, w1_v, in1_sem, _N_H1, _N_IN),
            (src3_h, w3_h, o3, acc3, src3_v, w3_v, in3_sem, _N_OUT, _N_H2),
        )

        in_dmas = []
        for (src_h, w_h, _, _, src_v, w_v, sem, n_nodes, _) in layers:
            n_e = (n_nodes // _NW) * _FAN
            base_e = wid * n_e
            in_dmas.append(pltpu.async_copy(
                src_h.at[pl.ds(base_e, n_e)], src_v, sem))
            in_dmas.append(pltpu.async_copy(
                w_h.at[pl.ds(base_e, n_e)], w_v, sem))

        for (_, _, _, acc, _, _, _, n_nodes, d) in layers:
            npw = n_nodes // _NW

            def zero_body(j, _, acc=acc, d=d):
                for c in range(d // _L):
                    acc[j, pl.ds(c * _L, _L)] = zeros16
                return 0
            lax.fori_loop(0, npw, zero_body, 0)

        out_dmas = []
        for li, (_, _, o_h, acc, src_v, w_v, _, n_nodes, d) in enumerate(
                layers):
            in_dmas[2 * li].wait()
            in_dmas[2 * li + 1].wait()
            npw = n_nodes // _NW
            nblocks = max(1, npw // _L)
            for nb in range(nblocks):
                local_nodes = lanes + nb * _L
                mask = local_nodes < npw if npw < _L else None
                for i in range(_FAN):
                    eidx = local_nodes * _FAN + i
                    cols = plsc.load_gather(src_v, [eidx])
                    vals = plsc.load_gather(w_v, [eidx])
                    if mask is None:
                        plsc.addupdate_scatter(acc, [local_nodes, cols], vals)
                    else:
                        plsc.addupdate_scatter(acc, [local_nodes, cols], vals,
                                               mask=mask)
                rows = min(_L, npw)
                out_dmas.append(pltpu.async_copy(
                    acc.at[pl.ds(nb * _L, rows)],
                    o_h.at[pl.ds(wid * npw + nb * _L, rows)], out_sem))
        for dma in out_dmas:
            dma.wait()

    return k(src1, w1, src2, w2, src3, w3)


def _tc_forward(x, w1t, w2t, w3t):
    bm = 1024
    dn = (((1,), (1,)), ((), ()))

    def body(x_ref, w1_ref, w2_ref, w3_ref, o_ref):
        h1 = jnp.tanh(lax.dot_general(x_ref[...], w1_ref[...], dn,
                                      preferred_element_type=jnp.float32))
        h2 = jnp.tanh(lax.dot_general(h1, w2_ref[...], dn,
                                      preferred_element_type=jnp.float32))
        o_ref[...] = jnp.tanh(lax.dot_general(h2, w3_ref[...], dn,
                                              preferred_element_type=jnp.float32))

    return pl.pallas_call(
        body,
        grid=(_BATCH // bm,),
        in_specs=[
            pl.BlockSpec((bm, _N_IN), lambda i: (i, 0)),
            pl.BlockSpec((_N_H1, _N_IN), lambda i: (0, 0)),
            pl.BlockSpec((_N_H2, _N_H1), lambda i: (0, 0)),
            pl.BlockSpec((_N_OUT, _N_H2), lambda i: (0, 0)),
        ],
        out_specs=pl.BlockSpec((bm, _N_OUT), lambda i: (i, 0)),
        out_shape=jax.ShapeDtypeStruct((_BATCH, _N_OUT), jnp.float32),
    )(x, w1t, w2t, w3t)


def kernel(x, w1, w2, w3, src1, dst1, src2, dst2, src3, dst3):
    del dst1, dst2, dst3
    w1t, w2t, w3t = _sc_densify(src1, w1, src2, w2, src3, w3)
    return _tc_forward(x, w1t, w2t, w3t)
